# baseline (speedup 1.0000x reference)
import functools

import jax
import jax.numpy as jnp
from jax import lax
from jax.experimental import pallas as pl
from jax.experimental.pallas import tpu as pltpu
from jax.experimental.pallas import tpu_sc as plsc

_VOCAB = 128
_D = 32
_NC = 2
_NS = 16
_NW = _NC * _NS
_L = 16
_R = 128
_CR = 8
_KG = 5
_KC = _CR - _KG


@functools.lru_cache(maxsize=None)
def _make_kernel(nrows: int):
  rows_w = nrows // _NW
  nch = rows_w // _CR
  assert nrows % _NW == 0 and rows_w % _CR == 0 and nch % 2 == 0

  mesh = plsc.VectorSubcoreMesh(core_axis_name="c", subcore_axis_name="s")

  @functools.partial(
      pl.kernel,
      out_type=jax.ShapeDtypeStruct((nrows, _R, _D), jnp.float32),
      mesh=mesh,
      compiler_params=pltpu.CompilerParams(
          needs_layout_passes=False, use_tc_tiling_on_sc=False),
      scratch_types=[
          pltpu.VMEM_SHARED((_VOCAB, _D), jnp.float32),
          pltpu.VMEM((_VOCAB, _D), jnp.float32),
          pltpu.VMEM((_CR, _R), jnp.int32),
          pltpu.VMEM((_CR, _R), jnp.int32),
          pltpu.VMEM((_KG, _R, _D), jnp.float32),
          pltpu.VMEM((_KG, _R, _D), jnp.float32),
          pltpu.VMEM((_KC, _R, _D), jnp.float32),
          pltpu.VMEM((_KC, _R, _D), jnp.float32),
          pltpu.SemaphoreType.DMA,
          pltpu.SemaphoreType.DMA,
          pltpu.SemaphoreType.DMA,
          pltpu.SemaphoreType.DMA,
          pltpu.SemaphoreType.DMA,
          pltpu.SemaphoreType.DMA,
          pltpu.SemaphoreType.DMA,
      ],
  )
  def emb(x_hbm, table_hbm, out_hbm,
          table_sh, table_v, iv0, iv1, gv0, gv1, cv0, cv1,
          si0, si1, sg, sog0, sog1, soc0, soc1):
    wid = lax.axis_index("s") * _NC + lax.axis_index("c")
    w_base = wid * rows_w
    ivs = (iv0, iv1)
    gvs = (gv0, gv1)
    cvs = (cv0, cv1)
    sis = (si0, si1)
    sogs = (sog0, sog1)
    socs = (soc0, soc1)

    pltpu.sync_copy(table_hbm, table_v)
    @pl.when(lax.axis_index("s") == 0)
    def _():
      pltpu.sync_copy(table_hbm, table_sh)
    plsc.subcore_barrier()

    for b in range(2):
      pltpu.async_copy(
          x_hbm.at[pl.ds(w_base + b * _CR, _CR)], ivs[b], sis[b])

    def outer(gi, carry):
      for b in range(2):
        g = gi * 2 + b
        iv, gv, cv = ivs[b], gvs[b], cvs[b]
        si, sog, soc = sis[b], sogs[b], socs[b]
        base = w_base + g * _CR

        pltpu.make_async_copy(x_hbm.at[pl.ds(w_base, _CR)], iv, si).wait()

        @pl.when(gi > 0)
        def _():
          pltpu.make_async_copy(
              gv, out_hbm.at[pl.ds(w_base, _KG)], sog).wait()
          pltpu.make_async_copy(
              cv, out_hbm.at[pl.ds(w_base, _KC)], soc).wait()

        gathers = [
            pltpu.async_copy(table_sh.at[iv.at[r]], gv.at[r], sg)
            for r in range(_KG)
        ]

        for rr in range(_KC):
          ivr = iv.at[_KG + rr]

          def cgroup(j, c2, ivr=ivr, rr=rr):
            rows = ivr[pl.ds(j * _L, _L)]
            for u in range(_L):
              row = rows[u]
              c = j * _L + u
              cv[rr, c, pl.ds(0, _L)] = table_v[row, pl.ds(0, _L)]
              cv[rr, c, pl.ds(_L, _L)] = table_v[row, pl.ds(_L, _L)]
            return c2

          lax.fori_loop(0, _R // _L, cgroup, 0, unroll=False)

        for h in gathers:
          h.wait()

        nxt = jnp.minimum(g + 2, nch - 1)
        pltpu.async_copy(x_hbm.at[pl.ds(w_base + nxt * _CR, _CR)], iv, si)
        pltpu.async_copy(gv, out_hbm.at[pl.ds(base, _KG)], sog)
        pltpu.async_copy(cv, out_hbm.at[pl.ds(base + _KG, _KC)], soc)
      return carry

    lax.fori_loop(0, nch // 2, outer, 0, unroll=False)

    for b in range(2):
      pltpu.make_async_copy(
          x_hbm.at[pl.ds(w_base, _CR)], ivs[b], sis[b]).wait()
      pltpu.make_async_copy(
          gvs[b], out_hbm.at[pl.ds(w_base, _KG)], sogs[b]).wait()
      pltpu.make_async_copy(
          cvs[b], out_hbm.at[pl.ds(w_base, _KC)], socs[b]).wait()

  return emb


def kernel(x, table):
  xf = x.reshape(-1, _R).astype(jnp.int32)
  out = _make_kernel(xf.shape[0])(xf, table)
  return out.reshape(*x.shape, _D)


# speedup vs baseline: 1.0665x; 1.0665x over previous
import functools

import jax
import jax.numpy as jnp
from jax import lax
from jax.experimental import pallas as pl
from jax.experimental.pallas import tpu as pltpu
from jax.experimental.pallas import tpu_sc as plsc

_VOCAB = 128
_D = 32
_NC = 2
_NS = 16
_NW = _NC * _NS
_L = 16
_R = 128
_CR = 8
_KG = 5
_KC = _CR - _KG


@functools.lru_cache(maxsize=None)
def _make_kernel(nrows: int):
  rows_w = nrows // _NW
  nch = rows_w // _CR
  assert nrows % _NW == 0 and rows_w % _CR == 0 and nch % 2 == 0

  mesh = plsc.VectorSubcoreMesh(core_axis_name="c", subcore_axis_name="s")

  @functools.partial(
      pl.kernel,
      out_type=jax.ShapeDtypeStruct((nrows, _R, _D), jnp.float32),
      mesh=mesh,
      compiler_params=pltpu.CompilerParams(
          needs_layout_passes=False, use_tc_tiling_on_sc=False),
      scratch_types=[
          pltpu.VMEM_SHARED((_VOCAB, _D), jnp.float32),
          pltpu.VMEM((_VOCAB, _D), jnp.float32),
          pltpu.VMEM((_CR, _R), jnp.int32),
          pltpu.VMEM((_CR, _R), jnp.int32),
          pltpu.VMEM((_KG, _R, _D), jnp.float32),
          pltpu.VMEM((_KG, _R, _D), jnp.float32),
          pltpu.VMEM((_KC, _R, _D), jnp.float32),
          pltpu.VMEM((_KC, _R, _D), jnp.float32),
          pltpu.SemaphoreType.DMA,
          pltpu.SemaphoreType.DMA,
          pltpu.SemaphoreType.DMA,
          pltpu.SemaphoreType.DMA,
          pltpu.SemaphoreType.DMA,
          pltpu.SemaphoreType.DMA,
          pltpu.SemaphoreType.DMA,
      ],
  )
  def emb(x_hbm, table_hbm, out_hbm,
          table_sh, table_v, iv0, iv1, gv0, gv1, cv0, cv1,
          si0, si1, sg, sog0, sog1, soc0, soc1):
    wid = lax.axis_index("s") * _NC + lax.axis_index("c")
    w_base = wid * rows_w
    ivs = (iv0, iv1)
    gvs = (gv0, gv1)
    cvs = (cv0, cv1)
    sis = (si0, si1)
    sogs = (sog0, sog1)
    socs = (soc0, soc1)

    pltpu.sync_copy(table_hbm, table_v)
    @pl.when(lax.axis_index("s") == 0)
    def _():
      pltpu.sync_copy(table_hbm, table_sh)
    plsc.subcore_barrier()

    for b in range(2):
      pltpu.async_copy(
          x_hbm.at[pl.ds(w_base + b * _CR, _CR)], ivs[b], sis[b])

    def outer(gi, carry):
      for b in range(2):
        g = gi * 2 + b
        iv, gv, cv = ivs[b], gvs[b], cvs[b]
        si, sog, soc = sis[b], sogs[b], socs[b]
        base = w_base + g * _CR

        pltpu.make_async_copy(x_hbm.at[pl.ds(w_base, _CR)], iv, si).wait()

        @pl.when(gi > 0)
        def _():
          pltpu.make_async_copy(
              gv, out_hbm.at[pl.ds(w_base, _KG)], sog).wait()
          pltpu.make_async_copy(
              cv, out_hbm.at[pl.ds(w_base, _KC)], soc).wait()

        gathers = [
            pltpu.async_copy(table_sh.at[iv.at[r]], gv.at[r], sg)
            for r in range(_KG)
        ]

        for rr in range(_KC):
          ivr = iv.at[_KG + rr]

          lanes = lax.broadcasted_iota(jnp.int32, (_L,), 0)

          def cgroup(j, c2, ivr=ivr, rr=rr):
            rows = ivr[pl.ds(j * _L, _L)]
            for u in range(_L):
              bc = lax.gather(
                  rows, jnp.full((_L, 1), u, jnp.int32),
                  dimension_numbers=lax.GatherDimensionNumbers(
                      offset_dims=(), collapsed_slice_dims=(0,),
                      start_index_map=(0,)),
                  slice_sizes=(1,),
                  mode=lax.GatherScatterMode.PROMISE_IN_BOUNDS)
              c = j * _L + u
              v0 = plsc.load_gather(table_v, [bc, lanes])
              v1 = plsc.load_gather(table_v, [bc, lanes + _L])
              cv[rr, c, pl.ds(0, _L)] = v0
              cv[rr, c, pl.ds(_L, _L)] = v1
            return c2

          lax.fori_loop(0, _R // _L, cgroup, 0, unroll=False)

        for h in gathers:
          h.wait()

        nxt = jnp.minimum(g + 2, nch - 1)
        pltpu.async_copy(x_hbm.at[pl.ds(w_base + nxt * _CR, _CR)], iv, si)
        pltpu.async_copy(gv, out_hbm.at[pl.ds(base, _KG)], sog)
        pltpu.async_copy(cv, out_hbm.at[pl.ds(base + _KG, _KC)], soc)
      return carry

    lax.fori_loop(0, nch // 2, outer, 0, unroll=False)

    for b in range(2):
      pltpu.make_async_copy(
          x_hbm.at[pl.ds(w_base, _CR)], ivs[b], sis[b]).wait()
      pltpu.make_async_copy(
          gvs[b], out_hbm.at[pl.ds(w_base, _KG)], sogs[b]).wait()
      pltpu.make_async_copy(
          cvs[b], out_hbm.at[pl.ds(w_base, _KC)], socs[b]).wait()

  return emb


def kernel(x, table):
  xf = x.reshape(-1, _R).astype(jnp.int32)
  out = _make_kernel(xf.shape[0])(xf, table)
  return out.reshape(*x.shape, _D)
